# Initial kernel scaffold; baseline (speedup 1.0000x reference)
import jax
import jax.numpy as jnp
from jax.experimental import pallas as pl


def kernel(predicted_locs, predicted_scores, boxes, labels, priors_cxcy):
    raise NotImplementedError("write your pallas kernel here")


# speedup vs baseline: 6.0056x; 6.0056x over previous
import jax
import jax.numpy as jnp
from jax.experimental import pallas as pl
from jax.experimental.pallas import tpu as pltpu

N_PRIORS = 24564
N_CLASSES = 81
N_OBJ = 16
THRESHOLD = 0.5
NEG_POS_RATIO = 3

P_BLK = 2048
NP = (N_PRIORS + P_BLK - 1) // P_BLK
N_PAD = NP * P_BLK
R = N_PAD // 128
INF_BITS = 0x7F800000
NEG_INF = float("-inf")


def _match_kernel(nb, boxes_sm, labels_sm, priors_ref, locs_ref,
                  label_ref, npos_ref, locsum_ref, pfeo_sm):
    b = pl.program_id(0)

    @pl.when(b == 0)
    def _init():
        locsum_ref[0, 0] = 0.0

    row_i = jax.lax.broadcasted_iota(jnp.int32, (R, 128), 0)
    lane_i = jax.lax.broadcasted_iota(jnp.int32, (R, 128), 1)
    prior_idx = row_i * 128 + lane_i
    pvalid = prior_idx < N_PRIORS

    px = priors_ref[0]
    py = priors_ref[1]
    pw = priors_ref[2]
    ph = priors_ref[3]
    area2 = (pw - px) * (ph - py)

    best_ov = jnp.full((R, 128), NEG_INF, jnp.float32)
    best_obj = jnp.zeros((R, 128), jnp.int32)
    for j in range(N_OBJ):
        bx1 = boxes_sm[b, j, 0]
        by1 = boxes_sm[b, j, 1]
        bx2 = boxes_sm[b, j, 2]
        by2 = boxes_sm[b, j, 3]
        ix = jnp.maximum(jnp.minimum(bx2, pw) - jnp.maximum(bx1, px), 0.0)
        iy = jnp.maximum(jnp.minimum(by2, ph) - jnp.maximum(by1, py), 0.0)
        inter = ix * iy
        union = ((bx2 - bx1) * (by2 - by1) + area2) - inter
        iou = jnp.where(pvalid, inter / union, NEG_INF)
        upd = iou > best_ov
        best_ov = jnp.where(upd, iou, best_ov)
        best_obj = jnp.where(upd, j, best_obj)
        mj = jnp.max(iou)
        pfeo_sm[j] = jnp.min(jnp.where(iou == mj, prior_idx, jnp.int32(2147483647)))

    ofep = best_obj
    ov = best_ov
    for j in range(N_OBJ):
        hit = prior_idx == pfeo_sm[j]
        ofep = jnp.where(hit, j, ofep)
        ov = jnp.where(hit, 1.0, ov)

    lab = jnp.zeros((R, 128), jnp.int32)
    for j in range(N_OBJ):
        lab = jnp.where(ofep == j, labels_sm[b, j], lab)
    lab = jnp.where(ov < THRESHOLD, 0, lab)
    lab = jnp.where(pvalid, lab, 0)
    label_ref[0] = lab
    npos_ref[b, 0] = jnp.sum((lab != 0).astype(jnp.float32))

    bcx = jnp.zeros((R, 128), jnp.float32)
    bcy = jnp.zeros((R, 128), jnp.float32)
    bw = jnp.zeros((R, 128), jnp.float32)
    bh = jnp.zeros((R, 128), jnp.float32)
    for j in range(N_OBJ):
        sel = ofep == j
        bx1 = boxes_sm[b, j, 0]
        by1 = boxes_sm[b, j, 1]
        bx2 = boxes_sm[b, j, 2]
        by2 = boxes_sm[b, j, 3]
        bcx = jnp.where(sel, (bx2 + bx1) / 2.0, bcx)
        bcy = jnp.where(sel, (by2 + by1) / 2.0, bcy)
        bw = jnp.where(sel, bx2 - bx1, bw)
        bh = jnp.where(sel, by2 - by1, bh)
    gx = (bcx - px) / (pw / 10.0)
    gy = (bcy - py) / (ph / 10.0)
    gw = jnp.log(bw / pw) * 5.0
    gh = jnp.log(bh / ph) * 5.0

    posm = lab != 0
    l1 = (jnp.abs(locs_ref[0, 0] - gx) + jnp.abs(locs_ref[0, 1] - gy)
          + jnp.abs(locs_ref[0, 2] - gw) + jnp.abs(locs_ref[0, 3] - gh))
    locsum_ref[0, 0] = locsum_ref[0, 0] + jnp.sum(jnp.where(posm, l1, 0.0))


def _conf_kernel(scores_ref, lab_ref, se_ref, shc_ref):
    s = scores_ref[0]
    lab = lab_ref[0, 0]
    m = jnp.max(s, axis=1, keepdims=True)
    sh = s - m
    se_ref[0, 0] = jnp.sum(jnp.exp(sh), axis=1, keepdims=True)
    cls_i = jax.lax.broadcasted_iota(jnp.int32, (P_BLK, N_CLASSES), 1)
    shc_ref[0, 0] = jnp.sum(jnp.where(cls_i == lab, sh, 0.0), axis=1,
                            keepdims=True)


def _mine_kernel(nb, npos_sm, locsum_sm, se_ref, shc_ref, lab_ref,
                 out_ref, acc_sm):
    b = pl.program_id(0)

    @pl.when(b == 0)
    def _init():
        acc_sm[0] = 0.0
        acc_sm[1] = 0.0

    row_i = jax.lax.broadcasted_iota(jnp.int32, (R, 128), 0)
    lane_i = jax.lax.broadcasted_iota(jnp.int32, (R, 128), 1)
    pvalid = (row_i * 128 + lane_i) < N_PRIORS

    conf = jnp.log(se_ref[0]) - shc_ref[0]
    lab = lab_ref[0]
    posm = lab != 0
    acc_sm[0] = acc_sm[0] + jnp.sum(jnp.where(posm, conf, 0.0))

    cfneg = jnp.where(pvalid, jnp.where(posm, 0.0, conf), -1.0)
    bits = jax.lax.bitcast_convert_type(cfneg, jnp.int32)
    kk = (npos_sm[b, 0] * NEG_POS_RATIO).astype(jnp.int32)

    def body(_, carry):
        lo, hi = carry
        mid = lo + jax.lax.div(hi - lo, 2)
        c = jnp.sum((bits > mid).astype(jnp.int32))
        small = c < kk
        return (jnp.where(small, lo, mid + 1), jnp.where(small, mid, hi))

    lo, _ = jax.lax.fori_loop(0, 31, body, (jnp.int32(0), jnp.int32(INF_BITS)))
    vk = jax.lax.bitcast_convert_type(lo, jnp.float32)
    gt = bits > lo
    c = jnp.sum(gt.astype(jnp.int32))
    sgt = jnp.sum(jnp.where(gt, cfneg, 0.0))
    hn = jnp.where(kk == 0, 0.0, sgt + (kk - c).astype(jnp.float32) * vk)
    acc_sm[1] = acc_sm[1] + hn

    @pl.when(b == nb - 1)
    def _fin():
        tp = jnp.float32(0.0)
        for i in range(nb):
            tp = tp + npos_sm[i, 0]
        out_ref[0, 0] = ((acc_sm[1] + acc_sm[0]) / tp
                         + locsum_sm[0, 0] / (tp * 4.0))


def kernel(predicted_locs, predicted_scores, boxes, labels, priors_cxcy):
    nb = predicted_locs.shape[0]
    pad = N_PAD - N_PRIORS
    locs_t = jnp.transpose(predicted_locs, (0, 2, 1))
    locs_t = jnp.pad(locs_t, ((0, 0), (0, 0), (0, pad))).reshape(nb, 4, R, 128)
    pri = jnp.pad(priors_cxcy.T, ((0, 0), (0, pad))).reshape(4, R, 128)
    labels32 = labels.astype(jnp.int32)

    label_rows, npos, locsum = pl.pallas_call(
        lambda *refs: _match_kernel(nb, *refs),
        grid=(nb,),
        in_specs=[
            pl.BlockSpec(memory_space=pltpu.SMEM),
            pl.BlockSpec(memory_space=pltpu.SMEM),
            pl.BlockSpec((4, R, 128), lambda b: (0, 0, 0)),
            pl.BlockSpec((1, 4, R, 128), lambda b: (b, 0, 0, 0)),
        ],
        out_specs=[
            pl.BlockSpec((1, R, 128), lambda b: (b, 0, 0)),
            pl.BlockSpec(memory_space=pltpu.SMEM),
            pl.BlockSpec(memory_space=pltpu.SMEM),
        ],
        out_shape=[
            jax.ShapeDtypeStruct((nb, R, 128), jnp.int32),
            jax.ShapeDtypeStruct((nb, 1), jnp.float32),
            jax.ShapeDtypeStruct((1, 1), jnp.float32),
        ],
        scratch_shapes=[pltpu.SMEM((N_OBJ,), jnp.int32)],
        compiler_params=pltpu.CompilerParams(
            dimension_semantics=("arbitrary",)),
    )(boxes, labels32, pri, locs_t)

    lab_cols = label_rows.reshape(nb, NP, P_BLK, 1)
    se_cols, shc_cols = pl.pallas_call(
        _conf_kernel,
        grid=(nb, NP),
        in_specs=[
            pl.BlockSpec((1, P_BLK, N_CLASSES), lambda b, p: (b, p, 0)),
            pl.BlockSpec((1, 1, P_BLK, 1), lambda b, p: (b, p, 0, 0)),
        ],
        out_specs=[
            pl.BlockSpec((1, 1, P_BLK, 1), lambda b, p: (b, p, 0, 0)),
            pl.BlockSpec((1, 1, P_BLK, 1), lambda b, p: (b, p, 0, 0)),
        ],
        out_shape=[
            jax.ShapeDtypeStruct((nb, NP, P_BLK, 1), jnp.float32),
            jax.ShapeDtypeStruct((nb, NP, P_BLK, 1), jnp.float32),
        ],
        compiler_params=pltpu.CompilerParams(
            dimension_semantics=("arbitrary", "arbitrary")),
    )(predicted_scores, lab_cols)

    out = pl.pallas_call(
        lambda *refs: _mine_kernel(nb, *refs),
        grid=(nb,),
        in_specs=[
            pl.BlockSpec(memory_space=pltpu.SMEM),
            pl.BlockSpec(memory_space=pltpu.SMEM),
            pl.BlockSpec((1, R, 128), lambda b: (b, 0, 0)),
            pl.BlockSpec((1, R, 128), lambda b: (b, 0, 0)),
            pl.BlockSpec((1, R, 128), lambda b: (b, 0, 0)),
        ],
        out_specs=pl.BlockSpec(memory_space=pltpu.SMEM),
        out_shape=jax.ShapeDtypeStruct((1, 1), jnp.float32),
        scratch_shapes=[pltpu.SMEM((2,), jnp.float32)],
        compiler_params=pltpu.CompilerParams(
            dimension_semantics=("arbitrary",)),
    )(npos, locsum, se_cols.reshape(nb, R, 128),
      shc_cols.reshape(nb, R, 128), label_rows)
    return out[0, 0]
